# Initial kernel scaffold; baseline (speedup 1.0000x reference)
import jax
import jax.numpy as jnp
from jax.experimental import pallas as pl


def kernel(ref_feats, pred_feats, W0, b0, W1, b1, ref_to_merged, pred_to_merged, edge_src, edge_dst, k_offsets, n_merged, stride):
    raise NotImplementedError("write your pallas kernel here")


# speedup vs baseline: 4.9413x; 4.9413x over previous
import functools

import jax
import jax.numpy as jnp
from jax import lax
from jax.experimental import pallas as pl
from jax.experimental.pallas import tpu as pltpu
from jax.experimental.pallas import tpu_sc as plsc

C = 32
K = 27
NC = 2
NS = 16
NW = NC * NS
CH = 128
SCH = 512


def _sc_gather(table, idx3, n_rows, width):
    n_sc = idx3.shape[0]
    t_max = (n_sc + NW - 1) // NW
    mesh = plsc.VectorSubcoreMesh(core_axis_name="c", subcore_axis_name="s")

    @functools.partial(
        pl.kernel,
        mesh=mesh,
        compiler_params=pltpu.CompilerParams(use_tc_tiling_on_sc=False),
        out_type=jax.ShapeDtypeStruct((n_rows, width), jnp.float32),
        scratch_types=[
            pltpu.VMEM((4, CH), jnp.int32),
            pltpu.VMEM((SCH, width), jnp.float32),
            pltpu.SemaphoreType.DMA,
        ],
    )
    def k(table_hbm, idx_hbm, out_hbm, iv, rows, sem):
        wid = lax.axis_index("s") * NC + lax.axis_index("c")

        def body(t, carry):
            j = wid + t * NW

            @pl.when(j < n_sc)
            def _():
                pltpu.sync_copy(idx_hbm.at[j], iv)
                descs = [
                    pltpu.async_copy(
                        table_hbm.at[iv.at[q]],
                        rows.at[pl.ds(q * CH, CH)],
                        sem,
                    )
                    for q in range(SCH // CH)
                ]
                for d in descs:
                    d.wait()
                pltpu.sync_copy(rows, out_hbm.at[pl.ds(j * SCH, SCH)])

            return carry

        lax.fori_loop(0, t_max, body, 0)

    return k(table, idx3)


def _conv_gemm(g3, w, b, n_merged, relu_mask, out_c, npad, bn):
    nb = npad // bn

    def body(nm_ref, g_ref, w_ref, b_ref, o_ref):
        i = pl.program_id(0)
        k = pl.program_id(1)
        part = jnp.dot(g_ref[0], w_ref[0], preferred_element_type=jnp.float32)

        @pl.when(k == 0)
        def _():
            o_ref[...] = part + b_ref[0]

        @pl.when(k > 0)
        def _():
            o_ref[...] = o_ref[...] + part

        if relu_mask:
            @pl.when(k == K - 1)
            def _():
                rows = i * bn + lax.broadcasted_iota(jnp.int32, (bn, 1), 0)
                o_ref[...] = jnp.where(
                    rows < nm_ref[0], jnp.maximum(o_ref[...], 0.0), 0.0
                )

    grid_spec = pltpu.PrefetchScalarGridSpec(
        num_scalar_prefetch=1,
        grid=(nb, K),
        in_specs=[
            pl.BlockSpec((1, bn, 2 * C), lambda i, k, nm: (k, i, 0)),
            pl.BlockSpec((1, 2 * C, out_c), lambda i, k, nm: (k, 0, 0)),
            pl.BlockSpec((1, out_c), lambda i, k, nm: (0, 0)),
        ],
        out_specs=pl.BlockSpec((bn, out_c), lambda i, k, nm: (i, 0)),
    )
    return pl.pallas_call(
        body,
        grid_spec=grid_spec,
        out_shape=jax.ShapeDtypeStruct((npad, out_c), jnp.float32),
    )(jnp.asarray(n_merged, jnp.int32).reshape(1), g3, w, b.reshape(1, -1))


def kernel(ref_feats, pred_feats, W0, b0, W1, b1, ref_to_merged,
           pred_to_merged, edge_src, edge_dst, k_offsets, n_merged, stride):
    n_ref = ref_feats.shape[0]
    n_pred = pred_feats.shape[0]
    n_max = n_ref + n_pred
    npad = ((n_max + SCH - 1) // SCH) * SCH
    e_tot = edge_src.shape[0]
    i32 = jnp.int32

    ref_to_merged = ref_to_merged.astype(i32)
    pred_to_merged = pred_to_merged.astype(i32)
    edge_src = edge_src.astype(i32)
    edge_dst = edge_dst.astype(i32)

    kf = jnp.asarray(k_offsets, dtype=i32)
    k_of_edge = (
        jnp.searchsorted(kf, jnp.arange(e_tot, dtype=i32), side="right") - 1
    ).astype(i32)
    sent = npad - 1
    nbr = jnp.full((K * npad,), sent, i32).at[
        k_of_edge * npad + edge_dst
    ].set(edge_src, indices_are_sorted=True, unique_indices=True)
    inv_ref = jnp.full((npad,), n_ref, i32).at[ref_to_merged].set(
        jnp.arange(n_ref, dtype=i32), unique_indices=True
    )
    inv_pred = jnp.full((npad,), n_pred, i32).at[pred_to_merged].set(
        jnp.arange(n_pred, dtype=i32), unique_indices=True
    )
    table0 = jnp.concatenate(
        [
            ref_feats,
            jnp.zeros((16, C), jnp.float32),
            pred_feats,
            jnp.zeros((16, C), jnp.float32),
        ]
    )
    poff = n_ref + 16
    idx0 = jnp.stack(
        [inv_ref[nbr], inv_pred[nbr] + poff], axis=1
    ).reshape(-1, 4, CH)
    idx1 = nbr.reshape(-1, 4, CH)
    np_pad = ((n_pred + SCH - 1) // SCH) * SCH
    idx2 = jnp.pad(pred_to_merged, (0, np_pad - n_pred)).reshape(-1, 4, CH)

    g0 = _sc_gather(table0, idx0, 2 * K * npad, C)
    g0 = g0.reshape(K, npad, 2 * C)
    h0 = _conv_gemm(g0, W0, b0, n_merged, True, 2 * C, npad, SCH)
    g1 = _sc_gather(h0, idx1, K * npad, 2 * C)
    g1 = g1.reshape(K, npad, 2 * C)
    h1 = _conv_gemm(g1, W1, b1, n_merged, False, C, npad, SCH)
    out = _sc_gather(h1, idx2, np_pad, C)
    return out[:n_pred]
